# baseline (speedup 1.0000x reference)
import jax
import jax.numpy as jnp
from jax import lax
from jax.experimental import pallas as pl
from jax.experimental.pallas import tpu as pltpu
from jax.experimental.pallas import tpu_sc as plsc

M_TOTAL = 4_000_000
B_TOTAL = 2_000_000
CAP_KW = 50000.0

NC = 2
NS = 16
NW = NC * NS
LANES = 16

GCHUNK = 8000
NGCH = B_TOTAL // GCHUNK
GMAX = -(-NGCH // NW)
FULL_W = NGCH - (NGCH // NW) * NW

MWORDS = M_TOTAL // 4
MCH = 8000
NMCH = MWORDS // MCH
FULL_M = NMCH - (NMCH // NS) * NS

WCH = 8000
NWCH = MWORDS // WCH
WMAX = -(-NWCH // NW)
FULL_C = NWCH - (NWCH // NW) * NW


def _sc_build_body(thr_hbm, ids_hbm, partials_out, maska_out, maskb_out,
                   gidx0_v, gidx1_v, gval0_v, gval1_v, widx0_v, widx1_v,
                   wval0_v, wval1_v, mbuf_v, accv, pvec,
                   shared, sem_i0, sem_i1, sem_g0, sem_g1, sem_c0, sem_c1):
    cid = lax.axis_index("c")
    sid = lax.axis_index("s")
    wid = sid * NC + cid

    gidx = (gidx0_v, gidx1_v)
    gval = (gval0_v, gval1_v)
    widx = (widx0_v, widx1_v)
    wval = (wval0_v, wval1_v)
    sem_i = (sem_i0, sem_i1)
    sem_g = (sem_g0, sem_g1)
    sem_c = (sem_c0, sem_c1)

    def zb(j, _c):
        mbuf_v[pl.ds(pl.multiple_of(j * LANES, LANES), LANES)] = (
            jnp.zeros((LANES,), jnp.int32))
        return _c
    lax.fori_loop(0, MCH // LANES, zb, 0, unroll=8)

    n_mch = jnp.where(sid < FULL_M, NMCH // NS + 1, NMCH // NS)

    def zchunk(k, _c):
        c = sid + NS * k
        pltpu.sync_copy(mbuf_v, shared.at[pl.ds(pl.multiple_of(c * MCH, 8), MCH)])
        return _c
    lax.fori_loop(0, n_mch, zchunk, 0)

    accv[...] = jnp.zeros((LANES,), jnp.float32)
    plsc.subcore_barrier()

    n_ch = jnp.where(wid < FULL_W, GMAX, GMAX - 1)

    def chunk_body(t, acc):
        g = wid + NW * t
        pltpu.sync_copy(ids_hbm.at[pl.ds(g * GCHUNK, GCHUNK)], gidx0_v)
        h = GCHUNK // 4
        gsems = (sem_g0, sem_g1, sem_i0, sem_i1)
        gats = [pltpu.async_copy(
                    thr_hbm.at[gidx0_v.at[pl.ds(pl.multiple_of(q * h, 8), h)]],
                    gval0_v.at[pl.ds(pl.multiple_of(q * h, 8), h)], gsems[q])
                for q in range(4)]

        def enc(j, _c):
            sl = pl.ds(pl.multiple_of(j * LANES, LANES), LANES)
            v = gidx0_v[sl]
            one = jnp.full((LANES,), 1, jnp.int32)
            zero = jnp.full((LANES,), 0, jnp.int32)
            p = (jnp.where(v >= MWORDS, one, zero)
                 + jnp.where(v >= 2 * MWORDS, one, zero)
                 + jnp.where(v >= 3 * MWORDS, one, zero))
            widx0_v[sl] = v - p * MWORDS
            wval0_v[sl] = lax.shift_left(one, p * 8)
            return _c
        lax.fori_loop(0, GCHUNK // LANES, enc, 0, unroll=8)
        hs = GCHUNK // 2
        sca0 = pltpu.async_copy(wval0_v.at[pl.ds(0, hs)],
                                shared.at[widx0_v.at[pl.ds(0, hs)]],
                                sem_c0, add=True)
        sca = pltpu.async_copy(wval0_v.at[pl.ds(pl.multiple_of(hs, 8), hs)],
                               shared.at[widx0_v.at[pl.ds(pl.multiple_of(hs, 8), hs)]],
                               sem_c1, add=True)

        for gh in gats:
            gh.wait()

        def accb(j, ab):
            a0, a1 = ab
            s0 = pl.ds(pl.multiple_of(2 * j * LANES, LANES), LANES)
            s1 = pl.ds(pl.multiple_of((2 * j + 1) * LANES, LANES), LANES)
            return (a0 + gval0_v[s0], a1 + gval0_v[s1])
        a0, a1 = lax.fori_loop(0, GCHUNK // (2 * LANES), accb, acc, unroll=4)
        sca0.wait()
        sca.wait()
        return (a0, a1)

    acc = lax.fori_loop(0, n_ch, chunk_body,
                        (jnp.zeros((LANES,), jnp.float32),
                         jnp.zeros((LANES,), jnp.float32)))
    accv[...] = acc[0] + acc[1]


    pvec[...] = accv[...]
    pltpu.sync_copy(pvec, partials_out.at[wid])

    plsc.subcore_barrier()

    def dchunk(k, _c):
        c = sid + NS * k
        pltpu.sync_copy(shared.at[pl.ds(pl.multiple_of(c * MCH, 8), MCH)], mbuf_v)
        @pl.when(cid == 0)
        def _():
            pltpu.sync_copy(mbuf_v, maska_out.at[pl.ds(pl.multiple_of(c * MCH, 8), MCH)])
        @pl.when(cid == 1)
        def _():
            pltpu.sync_copy(mbuf_v, maskb_out.at[pl.ds(pl.multiple_of(c * MCH, 8), MCH)])
        return _c
    lax.fori_loop(0, n_mch, dchunk, 0)


def _sc_combine_body(cur_hbm, part_hbm, maska_hbm, maskb_hbm, out_hbm,
                     cbuf0_v, cbuf1_v, wa_v, wb_v, pbuf_v,
                     sem_w, sem_l0, sem_l1, sem_o0, sem_o1):
    cid = lax.axis_index("c")
    sid = lax.axis_index("s")
    wid = sid * NC + cid

    cbuf = (cbuf0_v, cbuf1_v)
    sem_l = (sem_l0, sem_l1)
    sem_o = (sem_o0, sem_o1)

    pltpu.sync_copy(part_hbm, pbuf_v)
    tot16 = jnp.zeros((LANES,), jnp.float32)
    for w in range(NW):
        tot16 = tot16 + pbuf_v[w, pl.ds(0, LANES)]
    total = tot16[0]
    for i in range(1, LANES):
        total = total + tot16[i]
    totv = jnp.full((LANES,), 0.0, jnp.float32) + total
    f = jnp.where(totv > CAP_KW, CAP_KW / totv, 1.0)

    n_wc = jnp.where(wid < FULL_C, WMAX, WMAX - 1)

    for k in range(WMAX):
        @pl.when(k < n_wc)
        def _(k=k):
            wbase = pl.multiple_of((wid + NW * k) * WCH, 8)
            ha = pltpu.async_copy(maska_hbm.at[pl.ds(wbase, WCH)], wa_v,
                                  sem_w)
            hb = pltpu.async_copy(maskb_hbm.at[pl.ds(wbase, WCH)], wb_v,
                                  sem_w)
            lo_h = [None, None]
            st_h = [None, None]
            lo_h[0] = pltpu.async_copy(cur_hbm.at[pl.ds(wbase, WCH)],
                                       cbuf[0], sem_l[0])
            ha.wait()
            hb.wait()

            def orw(j, _c):
                sl = pl.ds(pl.multiple_of(j * LANES, LANES), LANES)
                wa_v[sl] = wa_v[sl] | wb_v[sl]
                return _c
            lax.fori_loop(0, WCH // LANES, orw, 0, unroll=8)

            for p in range(4):
                bp = p % 2
                if p + 1 < 4:
                    if st_h[(p + 1) % 2] is not None:
                        st_h[(p + 1) % 2].wait()
                    lo_h[(p + 1) % 2] = pltpu.async_copy(
                        cur_hbm.at[pl.ds(pl.multiple_of((p + 1) * MWORDS + wbase, 8), WCH)],
                        cbuf[(p + 1) % 2], sem_l[(p + 1) % 2])
                lo_h[bp].wait()
                bmask = jnp.full((LANES,), 0xFF << (8 * p), jnp.int32)

                def comb(j, _c, bp=bp, bmask=bmask):
                    sl = pl.ds(pl.multiple_of(j * LANES, LANES), LANES)
                    w = wa_v[sl] & bmask
                    cv = cbuf[bp][sl]
                    cbuf[bp][sl] = jnp.where(w != 0, cv * f, cv)
                    return _c
                lax.fori_loop(0, WCH // LANES, comb, 0, unroll=8)
                st_h[bp] = pltpu.async_copy(
                    cbuf[bp], out_hbm.at[pl.ds(pl.multiple_of(p * MWORDS + wbase, 8), WCH)],
                    sem_o[bp])
            st_h[0].wait()
            st_h[1].wait()


@jax.jit
def kernel(charger_current_now, charger_throughput_now_kw, charger_ids_children):
    ids1 = charger_ids_children.astype(jnp.int32)

    mesh = plsc.VectorSubcoreMesh(core_axis_name="c", subcore_axis_name="s",
                                  num_cores=NC, num_subcores=NS)

    build_k = pl.kernel(
        _sc_build_body,
        out_type=(jax.ShapeDtypeStruct((NW, LANES), jnp.float32),
                  jax.ShapeDtypeStruct((MWORDS,), jnp.int32),
                  jax.ShapeDtypeStruct((MWORDS,), jnp.int32)),
        mesh=mesh,
        scratch_types=[
            pltpu.VMEM((GCHUNK,), jnp.int32),
            pltpu.VMEM((GCHUNK,), jnp.int32),
            pltpu.VMEM((GCHUNK,), jnp.float32),
            pltpu.VMEM((GCHUNK,), jnp.float32),
            pltpu.VMEM((GCHUNK,), jnp.int32),
            pltpu.VMEM((GCHUNK,), jnp.int32),
            pltpu.VMEM((GCHUNK,), jnp.int32),
            pltpu.VMEM((GCHUNK,), jnp.int32),
            pltpu.VMEM((MCH,), jnp.int32),
            pltpu.VMEM((LANES,), jnp.float32),
            pltpu.VMEM((LANES,), jnp.float32),
            pltpu.VMEM_SHARED((MWORDS,), jnp.int32),
            pltpu.SemaphoreType.DMA,
            pltpu.SemaphoreType.DMA,
            pltpu.SemaphoreType.DMA,
            pltpu.SemaphoreType.DMA,
            pltpu.SemaphoreType.DMA,
            pltpu.SemaphoreType.DMA,
        ],
    )
    partials, maska, maskb = build_k(charger_throughput_now_kw, ids1)

    combine_k = pl.kernel(
        _sc_combine_body,
        out_type=jax.ShapeDtypeStruct((M_TOTAL,), jnp.float32),
        mesh=mesh,
        scratch_types=[
            pltpu.VMEM((WCH,), jnp.float32),
            pltpu.VMEM((WCH,), jnp.float32),
            pltpu.VMEM((WCH,), jnp.int32),
            pltpu.VMEM((WCH,), jnp.int32),
            pltpu.VMEM((NW, LANES), jnp.float32),
            pltpu.SemaphoreType.DMA,
            pltpu.SemaphoreType.DMA,
            pltpu.SemaphoreType.DMA,
            pltpu.SemaphoreType.DMA,
            pltpu.SemaphoreType.DMA,
        ],
    )
    return combine_k(charger_current_now, partials, maska, maskb)


# speedup vs baseline: 1.0074x; 1.0074x over previous
import jax
import jax.numpy as jnp
from jax import lax
from jax.experimental import pallas as pl
from jax.experimental.pallas import tpu as pltpu
from jax.experimental.pallas import tpu_sc as plsc

M_TOTAL = 4_000_000
B_TOTAL = 2_000_000
CAP_KW = 50000.0

NC = 2
NS = 16
NW = NC * NS
LANES = 16

GCHUNK = 8000
NGCH = B_TOTAL // GCHUNK
GMAX = -(-NGCH // NW)
FULL_W = NGCH - (NGCH // NW) * NW

MWORDS = M_TOTAL // 4
MCH = 8000
NMCH = MWORDS // MCH
FULL_M = NMCH - (NMCH // NS) * NS

WCH = 8000
NWCH = MWORDS // WCH
WMAX = -(-NWCH // NW)
FULL_C = NWCH - (NWCH // NW) * NW


def _sc_build_body(thr_hbm, ids_hbm, partials_out, maska_out, maskb_out,
                   gidx0_v, gval0_v, widx0_v, wval0_v, mbuf_v, accv, pvec,
                   shared, sem_g0, sem_g1, sem_g2, sem_g3, sem_c0):
    cid = lax.axis_index("c")
    sid = lax.axis_index("s")
    wid = sid * NC + cid

    def zb(j, _c):
        mbuf_v[pl.ds(pl.multiple_of(j * LANES, LANES), LANES)] = (
            jnp.zeros((LANES,), jnp.int32))
        return _c
    lax.fori_loop(0, MCH // LANES, zb, 0, unroll=8)

    n_mch = jnp.where(sid < FULL_M, NMCH // NS + 1, NMCH // NS)

    def zchunk(k, _c):
        c = sid + NS * k
        pltpu.sync_copy(mbuf_v, shared.at[pl.ds(pl.multiple_of(c * MCH, 8), MCH)])
        return _c
    lax.fori_loop(0, n_mch, zchunk, 0)

    accv[...] = jnp.zeros((LANES,), jnp.float32)
    plsc.subcore_barrier()

    n_ch = jnp.where(wid < FULL_W, GMAX, GMAX - 1)

    def chunk_body(t, acc):
        g = wid + NW * t
        pltpu.sync_copy(ids_hbm.at[pl.ds(g * GCHUNK, GCHUNK)], gidx0_v)
        h = GCHUNK // 4
        gsems = (sem_g0, sem_g1, sem_g2, sem_g3)
        gats = [pltpu.async_copy(
                    thr_hbm.at[gidx0_v.at[pl.ds(pl.multiple_of(q * h, 8), h)]],
                    gval0_v.at[pl.ds(pl.multiple_of(q * h, 8), h)], gsems[q])
                for q in range(4)]

        def enc(j, _c):
            sl = pl.ds(pl.multiple_of(j * LANES, LANES), LANES)
            v = gidx0_v[sl]
            one = jnp.full((LANES,), 1, jnp.int32)
            zero = jnp.full((LANES,), 0, jnp.int32)
            p = (jnp.where(v >= MWORDS, one, zero)
                 + jnp.where(v >= 2 * MWORDS, one, zero)
                 + jnp.where(v >= 3 * MWORDS, one, zero))
            widx0_v[sl] = v - p * MWORDS
            wval0_v[sl] = lax.shift_left(one, p * 8)
            return _c
        lax.fori_loop(0, GCHUNK // LANES, enc, 0, unroll=8)
        sca = pltpu.async_copy(wval0_v, shared.at[widx0_v], sem_c0, add=True)

        for gh in gats:
            gh.wait()

        def accb(j, ab):
            a0, a1 = ab
            s0 = pl.ds(pl.multiple_of(2 * j * LANES, LANES), LANES)
            s1 = pl.ds(pl.multiple_of((2 * j + 1) * LANES, LANES), LANES)
            return (a0 + gval0_v[s0], a1 + gval0_v[s1])
        a0, a1 = lax.fori_loop(0, GCHUNK // (2 * LANES), accb, acc, unroll=4)
        sca.wait()
        return (a0, a1)

    acc = lax.fori_loop(0, n_ch, chunk_body,
                        (jnp.zeros((LANES,), jnp.float32),
                         jnp.zeros((LANES,), jnp.float32)))
    accv[...] = acc[0] + acc[1]


    pvec[...] = accv[...]
    pltpu.sync_copy(pvec, partials_out.at[wid])

    plsc.subcore_barrier()

    def dchunk(k, _c):
        c = sid + NS * k
        pltpu.sync_copy(shared.at[pl.ds(pl.multiple_of(c * MCH, 8), MCH)], mbuf_v)
        @pl.when(cid == 0)
        def _():
            pltpu.sync_copy(mbuf_v, maska_out.at[pl.ds(pl.multiple_of(c * MCH, 8), MCH)])
        @pl.when(cid == 1)
        def _():
            pltpu.sync_copy(mbuf_v, maskb_out.at[pl.ds(pl.multiple_of(c * MCH, 8), MCH)])
        return _c
    lax.fori_loop(0, n_mch, dchunk, 0)


def _sc_combine_body(cur_hbm, part_hbm, maska_hbm, maskb_hbm, out_hbm,
                     cbuf0_v, cbuf1_v, wa_v, wb_v, pbuf_v,
                     sem_w, sem_l0, sem_l1, sem_o0, sem_o1):
    cid = lax.axis_index("c")
    sid = lax.axis_index("s")
    wid = sid * NC + cid

    cbuf = (cbuf0_v, cbuf1_v)
    sem_l = (sem_l0, sem_l1)
    sem_o = (sem_o0, sem_o1)

    pltpu.sync_copy(part_hbm, pbuf_v)
    tot16 = jnp.zeros((LANES,), jnp.float32)
    for w in range(NW):
        tot16 = tot16 + pbuf_v[w, pl.ds(0, LANES)]
    total = tot16[0]
    for i in range(1, LANES):
        total = total + tot16[i]
    totv = jnp.full((LANES,), 0.0, jnp.float32) + total
    f = jnp.where(totv > CAP_KW, CAP_KW / totv, 1.0)

    n_wc = jnp.where(wid < FULL_C, WMAX, WMAX - 1)

    for k in range(WMAX):
        @pl.when(k < n_wc)
        def _(k=k):
            wbase = pl.multiple_of((wid + NW * k) * WCH, 8)
            ha = pltpu.async_copy(maska_hbm.at[pl.ds(wbase, WCH)], wa_v,
                                  sem_w)
            hb = pltpu.async_copy(maskb_hbm.at[pl.ds(wbase, WCH)], wb_v,
                                  sem_w)
            lo_h = [None, None]
            st_h = [None, None]
            lo_h[0] = pltpu.async_copy(cur_hbm.at[pl.ds(wbase, WCH)],
                                       cbuf[0], sem_l[0])
            ha.wait()
            hb.wait()

            def orw(j, _c):
                sl = pl.ds(pl.multiple_of(j * LANES, LANES), LANES)
                wa_v[sl] = wa_v[sl] | wb_v[sl]
                return _c
            lax.fori_loop(0, WCH // LANES, orw, 0, unroll=8)

            for p in range(4):
                bp = p % 2
                if p + 1 < 4:
                    if st_h[(p + 1) % 2] is not None:
                        st_h[(p + 1) % 2].wait()
                    lo_h[(p + 1) % 2] = pltpu.async_copy(
                        cur_hbm.at[pl.ds(pl.multiple_of((p + 1) * MWORDS + wbase, 8), WCH)],
                        cbuf[(p + 1) % 2], sem_l[(p + 1) % 2])
                lo_h[bp].wait()
                bmask = jnp.full((LANES,), 0xFF << (8 * p), jnp.int32)

                def comb(j, _c, bp=bp, bmask=bmask):
                    sl = pl.ds(pl.multiple_of(j * LANES, LANES), LANES)
                    w = wa_v[sl] & bmask
                    cv = cbuf[bp][sl]
                    cbuf[bp][sl] = jnp.where(w != 0, cv * f, cv)
                    return _c
                lax.fori_loop(0, WCH // LANES, comb, 0, unroll=8)
                st_h[bp] = pltpu.async_copy(
                    cbuf[bp], out_hbm.at[pl.ds(pl.multiple_of(p * MWORDS + wbase, 8), WCH)],
                    sem_o[bp])
            st_h[0].wait()
            st_h[1].wait()


@jax.jit
def kernel(charger_current_now, charger_throughput_now_kw, charger_ids_children):
    ids1 = charger_ids_children.astype(jnp.int32)

    mesh = plsc.VectorSubcoreMesh(core_axis_name="c", subcore_axis_name="s",
                                  num_cores=NC, num_subcores=NS)

    build_k = pl.kernel(
        _sc_build_body,
        out_type=(jax.ShapeDtypeStruct((NW, LANES), jnp.float32),
                  jax.ShapeDtypeStruct((MWORDS,), jnp.int32),
                  jax.ShapeDtypeStruct((MWORDS,), jnp.int32)),
        mesh=mesh,
        scratch_types=[
            pltpu.VMEM((GCHUNK,), jnp.int32),
            pltpu.VMEM((GCHUNK,), jnp.float32),
            pltpu.VMEM((GCHUNK,), jnp.int32),
            pltpu.VMEM((GCHUNK,), jnp.int32),
            pltpu.VMEM((MCH,), jnp.int32),
            pltpu.VMEM((LANES,), jnp.float32),
            pltpu.VMEM((LANES,), jnp.float32),
            pltpu.VMEM_SHARED((MWORDS,), jnp.int32),
            pltpu.SemaphoreType.DMA,
            pltpu.SemaphoreType.DMA,
            pltpu.SemaphoreType.DMA,
            pltpu.SemaphoreType.DMA,
            pltpu.SemaphoreType.DMA,
        ],
    )
    partials, maska, maskb = build_k(charger_throughput_now_kw, ids1)

    combine_k = pl.kernel(
        _sc_combine_body,
        out_type=jax.ShapeDtypeStruct((M_TOTAL,), jnp.float32),
        mesh=mesh,
        scratch_types=[
            pltpu.VMEM((WCH,), jnp.float32),
            pltpu.VMEM((WCH,), jnp.float32),
            pltpu.VMEM((WCH,), jnp.int32),
            pltpu.VMEM((WCH,), jnp.int32),
            pltpu.VMEM((NW, LANES), jnp.float32),
            pltpu.SemaphoreType.DMA,
            pltpu.SemaphoreType.DMA,
            pltpu.SemaphoreType.DMA,
            pltpu.SemaphoreType.DMA,
            pltpu.SemaphoreType.DMA,
        ],
    )
    return combine_k(charger_current_now, partials, maska, maskb)
